# baseline (speedup 1.0000x reference)
import functools

import jax
import jax.numpy as jnp
from jax import lax
from jax.experimental import pallas as pl
from jax.experimental.pallas import tpu as pltpu
from jax.experimental.pallas import tpu_sc as plsc

_R, _C = 64, 8192
_K = 128
_L = 16
_NV = _C // _L
_NCORE, _NSUB = 2, 16
_NW = _NCORE * _NSUB
_ROWS_PER_W = _R // _NW
_CL = 513
_INV_SQRT2 = 0.7071067811865476


def _splat_i32(v):
    return jnp.full((_L,), v, jnp.int32)


def _keys(xi):
    b = plsc.bitcast(xi, jnp.int32)
    return b ^ lax.shift_right_logical(b >> 31, 1)


def _process_row(xrow, orow, hist, cumh, cx):
    lane = lax.iota(jnp.int32, _L)
    lane_base = lane * 257
    ones = jnp.ones((_L,), jnp.int32)
    zeros_i = jnp.zeros((_L,), jnp.int32)
    zeros_f = jnp.zeros((_L,), jnp.float32)

    with jax.named_scope("ph_zero"):
        @plsc.parallel_loop(0, 257, unroll=8)
        def _zero_body(i):
            hist[pl.ds(pl.multiple_of(i * _L, _L), _L)] = zeros_i

    lane_base128 = lane_base + 128
    with jax.named_scope("ph_hist"):
        @plsc.parallel_loop(0, _NV, unroll=8)
        def _hist_body(v):
            xi = xrow[pl.ds(pl.multiple_of(v * _L, _L), _L)]
            plsc.addupdate_scatter(
                hist, [(_keys(xi) >> 24) + lane_base128], ones)

    scan_scope = jax.named_scope("ph_scan")
    scan_scope.__enter__()
    carry = zeros_i
    for c in range(16):
        tot = hist[pl.ds(c * _L, _L)]
        for l in range(1, 16):
            tot = tot + hist[pl.ds(l * 257 + c * _L, _L)]
        pc = plsc.cumsum(tot) + carry
        cumh[pl.ds(c * _L, _L)] = pc
        carry = plsc.load_gather(cumh, [_splat_i32(c * _L + 15)])

    target = _splat_i32(_C - _K)
    b0 = zeros_i
    for c in range(16):
        pc = cumh[pl.ds(c * _L, _L)]
        m = pc <= target
        if c == 15:
            m = m & (lane < 15)
        b0 = b0 + plsc.all_reduce_population_count(m)
    p_b0 = plsc.load_gather(cumh, [b0])
    count_gt_b0 = _splat_i32(_C) - p_b0
    r0 = _splat_i32(_K) - count_gt_b0
    scan_scope.__exit__(None, None, None)

    lane_cbase = lane * _CL
    lo_key = (b0 - 128) << 24
    hi_key = jnp.where(b0 == 255, jnp.int32(0x7F800000),
                       lo_key + jnp.int32(0x01000000))

    def _compact_body(v, st):
        pos_l, s_above = st
        xi = xrow[pl.ds(pl.multiple_of(v * _L, _L), _L)]
        key = _keys(xi)
        m_gt = key >= hi_key
        s_above = s_above + jnp.where(m_gt, xi, 0.0)
        m_in = (key >= lo_key) & (~m_gt)
        plsc.store_scatter(cx, [lane_cbase + pos_l], xi, mask=m_in)
        pos_l = pos_l + jnp.where(m_in, 1, 0)
        return pos_l, s_above

    with jax.named_scope("ph_compact"):
        pos_l, sum_above = plsc.parallel_loop(
            0, _NV, unroll=8, carry=(zeros_i, zeros_f))(_compact_body)
    nj = jnp.max(pos_l)

    tail_scope = jax.named_scope("ph_tail")
    tail_scope.__enter__()
    low_mask = jnp.int32(0x00FFFFFF)
    prefix = zeros_i
    for bit in range(23, -1, -1):
        t = prefix | (1 << bit)

        def _cnt_body(j, cnt, t=t):
            xi = plsc.load_gather(cx, [lane_cbase + j])
            valid = pos_l > j
            m = ((_keys(xi) & low_mask) >= t) & valid
            return cnt + jnp.where(m, 1, 0)

        cnt_v = plsc.parallel_loop(0, nj, unroll=4, carry=zeros_i)(_cnt_body)
        cnt = jnp.broadcast_to(jnp.sum(cnt_v), (_L,))
        prefix = jnp.where(cnt >= r0, t, prefix)

    def _stats_body(j, st):
        c_gt, c_eq, s_gt = st
        xi = plsc.load_gather(cx, [lane_cbase + j])
        valid = pos_l > j
        lowv = _keys(xi) & low_mask
        m_gt = (lowv > prefix) & valid
        m_eq = (lowv == prefix) & valid
        c_gt = c_gt + jnp.where(m_gt, 1, 0)
        c_eq = c_eq + jnp.where(m_eq, 1, 0)
        s_gt = s_gt + jnp.where(m_gt, xi, 0.0)
        return c_gt, c_eq, s_gt

    c_gt_v, c_eq_v, s_gt_c = plsc.parallel_loop(
        0, nj, unroll=4, carry=(zeros_i, zeros_i, zeros_f))(_stats_body)
    c_gt_c = jnp.broadcast_to(jnp.sum(c_gt_v), (_L,))
    c_eq = jnp.broadcast_to(jnp.sum(c_eq_v), (_L,))
    tail_scope.__exit__(None, None, None)

    count_gt = count_gt_b0 + c_gt_c
    thresh_key = lo_key | prefix
    tbits = jnp.where(thresh_key < 0, thresh_key ^ jnp.int32(0x7FFFFFFF),
                      thresh_key)
    thresh_val = plsc.bitcast(tbits, jnp.float32)

    sum_gt_vec = sum_above + s_gt_c
    sum_gt = jnp.broadcast_to(jnp.sum(sum_gt_vec), (_L,))
    mag = sum_gt + (_splat_i32(_K) - count_gt).astype(jnp.float32) * thresh_val
    denom = mag - mag * jnp.float32(_INV_SQRT2) + jnp.float32(1e-7)
    inv = jnp.float32(1.0) / denom
    inv = jnp.where(jnp.abs(inv) == jnp.inf, 0.0, inv)

    with jax.named_scope("ph_out"):
        @plsc.parallel_loop(0, _NV, unroll=8)
        def _out_body(v):
            xi = xrow[pl.ds(pl.multiple_of(v * _L, _L), _L)]
            orow[pl.ds(pl.multiple_of(v * _L, _L), _L)] = jnp.where(
                xi >= thresh_val, xi * inv, 0.0)

    r_eq = _splat_i32(_K) - count_gt

    @pl.when(jnp.max(c_eq) > jnp.max(r_eq))
    def _tie_fix():
        def _fix_body(v, seen):
            xi = xrow[pl.ds(pl.multiple_of(v * _L, _L), _L)]
            m_eq = _keys(xi) == thresh_key
            mi = jnp.where(m_eq, 1, 0).astype(jnp.int32)
            rank = seen + plsc.cumsum(mi) - mi
            kill = m_eq & (rank >= r_eq)
            ov = orow[pl.ds(pl.multiple_of(v * _L, _L), _L)]
            orow[pl.ds(pl.multiple_of(v * _L, _L), _L)] = jnp.where(
                kill, 0.0, ov)
            return seen + plsc.all_reduce_population_count(m_eq)

        lax.fori_loop(0, _NV, _fix_body, zeros_i)


def _make_kernel():
    mesh = plsc.VectorSubcoreMesh(core_axis_name="c", subcore_axis_name="s")

    @functools.partial(
        pl.kernel,
        out_type=jax.ShapeDtypeStruct((_R, _C), jnp.float32),
        mesh=mesh,
        compiler_params=pltpu.CompilerParams(needs_layout_passes=False),
        scratch_types=[
            pltpu.VMEM((_C,), jnp.float32),
            pltpu.VMEM((_C,), jnp.float32),
            pltpu.VMEM((_C,), jnp.float32),
            pltpu.VMEM((_C,), jnp.float32),
            pltpu.VMEM((16 * 257,), jnp.int32),
            pltpu.VMEM((256,), jnp.int32),
            pltpu.VMEM((_L * _CL,), jnp.float32),
            pltpu.SemaphoreType.DMA,
            pltpu.SemaphoreType.DMA,
            pltpu.SemaphoreType.DMA,
            pltpu.SemaphoreType.DMA,
        ],
    )
    def _remaxk(x_hbm, out_hbm, xrow0, xrow1, orow0, orow1, hist, cumh,
                cx, sin0, sin1, sout0, sout1):
        wid = lax.axis_index("s") * _NCORE + lax.axis_index("c")
        row0 = wid * _ROWS_PER_W

        cp0 = pltpu.async_copy(x_hbm.at[row0], xrow0, sin0)
        cp1 = pltpu.async_copy(x_hbm.at[row0 + 1], xrow1, sin1)
        cp0.wait()
        _process_row(xrow0, orow0, hist, cumh, cx)
        w0 = pltpu.async_copy(orow0, out_hbm.at[row0], sout0)
        cp1.wait()
        _process_row(xrow1, orow1, hist, cumh, cx)
        w1 = pltpu.async_copy(orow1, out_hbm.at[row0 + 1], sout1)
        w0.wait()
        w1.wait()

    return _remaxk


_remaxk_kernel = _make_kernel()


@jax.jit
def kernel(x):
    return _remaxk_kernel(x)


# speedup vs baseline: 1.1413x; 1.1413x over previous
import functools

import jax
import jax.numpy as jnp
from jax import lax
from jax.experimental import pallas as pl
from jax.experimental.pallas import tpu as pltpu
from jax.experimental.pallas import tpu_sc as plsc

_R, _C = 64, 8192
_K = 128
_L = 16
_NV = _C // _L
_NCORE, _NSUB = 2, 16
_NW = _NCORE * _NSUB
_ROWS_PER_W = _R // _NW
_CL = 513
_INV_SQRT2 = 0.7071067811865476


def _splat_i32(v):
    return jnp.full((_L,), v, jnp.int32)


def _keys(xi):
    b = plsc.bitcast(xi, jnp.int32)
    return b ^ lax.shift_right_logical(b >> 31, 1)


def _process_row(xrow, orow, hist, cumh, cx):
    lane = lax.iota(jnp.int32, _L)
    lane_base = lane * 257
    ones = jnp.ones((_L,), jnp.int32)
    zeros_i = jnp.zeros((_L,), jnp.int32)
    zeros_f = jnp.zeros((_L,), jnp.float32)

    with jax.named_scope("ph_zero"):
        @plsc.parallel_loop(0, 257, unroll=8)
        def _zero_body(i):
            hist[pl.ds(pl.multiple_of(i * _L, _L), _L)] = zeros_i

    lane_base128 = lane_base + 128
    with jax.named_scope("ph_hist"):
        @plsc.parallel_loop(0, _NV, unroll=8)
        def _hist_body(v):
            xi = xrow[pl.ds(pl.multiple_of(v * _L, _L), _L)]
            plsc.addupdate_scatter(
                hist, [(_keys(xi) >> 24) + lane_base128], ones)

    scan_scope = jax.named_scope("ph_scan")
    scan_scope.__enter__()
    carry = zeros_i
    for c in range(16):
        tot = hist[pl.ds(c * _L, _L)]
        for l in range(1, 16):
            tot = tot + hist[pl.ds(l * 257 + c * _L, _L)]
        pc = plsc.cumsum(tot) + carry
        cumh[pl.ds(c * _L, _L)] = pc
        carry = plsc.load_gather(cumh, [_splat_i32(c * _L + 15)])

    target = _splat_i32(_C - _K)
    b0 = zeros_i
    for c in range(16):
        pc = cumh[pl.ds(c * _L, _L)]
        m = pc <= target
        if c == 15:
            m = m & (lane < 15)
        b0 = b0 + plsc.all_reduce_population_count(m)
    p_b0 = plsc.load_gather(cumh, [b0])
    count_gt_b0 = _splat_i32(_C) - p_b0
    r0 = _splat_i32(_K) - count_gt_b0
    scan_scope.__exit__(None, None, None)

    lo_key = (b0 - 128) << 24
    hi_key = jnp.where(b0 == 255, jnp.int32(0x7F800000),
                       lo_key + jnp.int32(0x01000000))

    def _compact_body(v, st):
        pos, s_above = st
        xi = xrow[pl.ds(pl.multiple_of(v * _L, _L), _L)]
        key = _keys(xi)
        m_gt = key >= hi_key
        s_above = s_above + jnp.where(m_gt, xi, 0.0)
        m_in = (key >= lo_key) & (~m_gt)
        mi = jnp.where(m_in, 1, 0)
        plsc.store_scatter(cx, [pos + plsc.cumsum(mi) - mi], xi, mask=m_in)
        pos = pos + plsc.all_reduce_population_count(m_in)
        return pos, s_above

    with jax.named_scope("ph_compact"):
        pos_l, sum_above = plsc.parallel_loop(
            0, _NV, unroll=8, carry=(zeros_i, zeros_f))(_compact_body)
    n_cand = pos_l
    nj = (jnp.max(pos_l) + _L - 1) // _L

    tail_scope = jax.named_scope("ph_tail")
    tail_scope.__enter__()
    low_mask = jnp.int32(0x00FFFFFF)
    prefix = zeros_i
    for bit in range(23, -1, -1):
        t = prefix | (1 << bit)

        def _cnt_body(j, cnt, t=t):
            xi = cx[pl.ds(pl.multiple_of(j * _L, _L), _L)]
            valid = (j * _L + lane) < n_cand
            m = ((_keys(xi) & low_mask) >= t) & valid
            return cnt + jnp.where(m, 1, 0)

        cnt_v = lax.fori_loop(0, nj, _cnt_body, zeros_i)
        cnt = jnp.broadcast_to(jnp.sum(cnt_v), (_L,))
        prefix = jnp.where(cnt >= r0, t, prefix)

    def _stats_body(j, st):
        c_gt, c_eq, s_gt = st
        xi = cx[pl.ds(pl.multiple_of(j * _L, _L), _L)]
        valid = (j * _L + lane) < n_cand
        lowv = _keys(xi) & low_mask
        m_gt = (lowv > prefix) & valid
        m_eq = (lowv == prefix) & valid
        c_gt = c_gt + jnp.where(m_gt, 1, 0)
        c_eq = c_eq + jnp.where(m_eq, 1, 0)
        s_gt = s_gt + jnp.where(m_gt, xi, 0.0)
        return c_gt, c_eq, s_gt

    c_gt_v, c_eq_v, s_gt_c = lax.fori_loop(
        0, nj, _stats_body, (zeros_i, zeros_i, zeros_f))
    c_gt_c = jnp.broadcast_to(jnp.sum(c_gt_v), (_L,))
    c_eq = jnp.broadcast_to(jnp.sum(c_eq_v), (_L,))
    tail_scope.__exit__(None, None, None)

    count_gt = count_gt_b0 + c_gt_c
    thresh_key = lo_key | prefix
    tbits = jnp.where(thresh_key < 0, thresh_key ^ jnp.int32(0x7FFFFFFF),
                      thresh_key)
    thresh_val = plsc.bitcast(tbits, jnp.float32)

    sum_gt_vec = sum_above + s_gt_c
    sum_gt = jnp.broadcast_to(jnp.sum(sum_gt_vec), (_L,))
    mag = sum_gt + (_splat_i32(_K) - count_gt).astype(jnp.float32) * thresh_val
    denom = mag - mag * jnp.float32(_INV_SQRT2) + jnp.float32(1e-7)
    inv = jnp.float32(1.0) / denom
    inv = jnp.where(jnp.abs(inv) == jnp.inf, 0.0, inv)

    with jax.named_scope("ph_out"):
        @plsc.parallel_loop(0, _NV, unroll=8)
        def _out_body(v):
            xi = xrow[pl.ds(pl.multiple_of(v * _L, _L), _L)]
            orow[pl.ds(pl.multiple_of(v * _L, _L), _L)] = jnp.where(
                xi >= thresh_val, xi * inv, 0.0)

    r_eq = _splat_i32(_K) - count_gt

    @pl.when(jnp.max(c_eq) > jnp.max(r_eq))
    def _tie_fix():
        def _fix_body(v, seen):
            xi = xrow[pl.ds(pl.multiple_of(v * _L, _L), _L)]
            m_eq = _keys(xi) == thresh_key
            mi = jnp.where(m_eq, 1, 0).astype(jnp.int32)
            rank = seen + plsc.cumsum(mi) - mi
            kill = m_eq & (rank >= r_eq)
            ov = orow[pl.ds(pl.multiple_of(v * _L, _L), _L)]
            orow[pl.ds(pl.multiple_of(v * _L, _L), _L)] = jnp.where(
                kill, 0.0, ov)
            return seen + plsc.all_reduce_population_count(m_eq)

        lax.fori_loop(0, _NV, _fix_body, zeros_i)


def _make_kernel():
    mesh = plsc.VectorSubcoreMesh(core_axis_name="c", subcore_axis_name="s")

    @functools.partial(
        pl.kernel,
        out_type=jax.ShapeDtypeStruct((_R, _C), jnp.float32),
        mesh=mesh,
        compiler_params=pltpu.CompilerParams(needs_layout_passes=False),
        scratch_types=[
            pltpu.VMEM((_C,), jnp.float32),
            pltpu.VMEM((_C,), jnp.float32),
            pltpu.VMEM((_C,), jnp.float32),
            pltpu.VMEM((_C,), jnp.float32),
            pltpu.VMEM((16 * 257,), jnp.int32),
            pltpu.VMEM((256,), jnp.int32),
            pltpu.VMEM((_C,), jnp.float32),
            pltpu.SemaphoreType.DMA,
            pltpu.SemaphoreType.DMA,
            pltpu.SemaphoreType.DMA,
            pltpu.SemaphoreType.DMA,
        ],
    )
    def _remaxk(x_hbm, out_hbm, xrow0, xrow1, orow0, orow1, hist, cumh,
                cx, sin0, sin1, sout0, sout1):
        wid = lax.axis_index("s") * _NCORE + lax.axis_index("c")
        row0 = wid * _ROWS_PER_W

        cp0 = pltpu.async_copy(x_hbm.at[row0], xrow0, sin0)
        cp1 = pltpu.async_copy(x_hbm.at[row0 + 1], xrow1, sin1)
        cp0.wait()
        _process_row(xrow0, orow0, hist, cumh, cx)
        w0 = pltpu.async_copy(orow0, out_hbm.at[row0], sout0)
        cp1.wait()
        _process_row(xrow1, orow1, hist, cumh, cx)
        w1 = pltpu.async_copy(orow1, out_hbm.at[row0 + 1], sout1)
        w0.wait()
        w1.wait()

    return _remaxk


_remaxk_kernel = _make_kernel()


@jax.jit
def kernel(x):
    return _remaxk_kernel(x)


# speedup vs baseline: 1.2736x; 1.1159x over previous
import functools

import jax
import jax.numpy as jnp
from jax import lax
from jax.experimental import pallas as pl
from jax.experimental.pallas import tpu as pltpu
from jax.experimental.pallas import tpu_sc as plsc

_R, _C = 64, 8192
_K = 128
_L = 16
_NV = _C // _L
_NCORE, _NSUB = 2, 16
_NW = _NCORE * _NSUB
_ROWS_PER_W = _R // _NW
_CL = 513
_INV_SQRT2 = 0.7071067811865476


def _splat_i32(v):
    return jnp.full((_L,), v, jnp.int32)


def _keys(xi):
    b = plsc.bitcast(xi, jnp.int32)
    return b ^ lax.shift_right_logical(b >> 31, 1)


def _process_row(xrow, orow, hist, cumh, cx):
    lane = lax.iota(jnp.int32, _L)
    lane_base = lane * 257
    ones = jnp.ones((_L,), jnp.int32)
    zeros_i = jnp.zeros((_L,), jnp.int32)
    zeros_f = jnp.zeros((_L,), jnp.float32)

    with jax.named_scope("ph_zero"):
        @plsc.parallel_loop(0, 257, unroll=8)
        def _zero_body(i):
            hist[pl.ds(pl.multiple_of(i * _L, _L), _L)] = zeros_i

    lane_base128 = lane_base + 128
    with jax.named_scope("ph_hist"):
        @plsc.parallel_loop(0, _NV, unroll=8)
        def _hist_body(v):
            xi = xrow[pl.ds(pl.multiple_of(v * _L, _L), _L)]
            plsc.addupdate_scatter(
                hist, [(_keys(xi) >> 24) + lane_base128], ones)

    scan_scope = jax.named_scope("ph_scan")
    scan_scope.__enter__()
    carry = zeros_i
    for c in range(16):
        tot = hist[pl.ds(c * _L, _L)]
        for l in range(1, 16):
            tot = tot + hist[pl.ds(l * 257 + c * _L, _L)]
        pc = plsc.cumsum(tot) + carry
        cumh[pl.ds(c * _L, _L)] = pc
        carry = plsc.load_gather(cumh, [_splat_i32(c * _L + 15)])

    target = _splat_i32(_C - _K)
    b0 = zeros_i
    for c in range(16):
        pc = cumh[pl.ds(c * _L, _L)]
        m = pc <= target
        if c == 15:
            m = m & (lane < 15)
        b0 = b0 + plsc.all_reduce_population_count(m)
    p_b0 = plsc.load_gather(cumh, [b0])
    count_gt_b0 = _splat_i32(_C) - p_b0
    r0 = _splat_i32(_K) - count_gt_b0
    scan_scope.__exit__(None, None, None)

    lo_key = (b0 - 128) << 24
    hi_key = jnp.where(b0 == 255, jnp.int32(0x7F800000),
                       lo_key + jnp.int32(0x01000000))

    def _compact_body(v, st):
        pos, s_above = st
        xi = xrow[pl.ds(pl.multiple_of(v * _L, _L), _L)]
        key = _keys(xi)
        m_gt = key >= hi_key
        s_above = s_above + jnp.where(m_gt, xi, 0.0)
        m_in = (key >= lo_key) & (~m_gt)
        mi = jnp.where(m_in, 1, 0)
        plsc.store_scatter(cx, [pos + plsc.cumsum(mi) - mi], xi, mask=m_in)
        pos = pos + plsc.all_reduce_population_count(m_in)
        return pos, s_above

    with jax.named_scope("ph_compact"):
        pos_l, sum_above = plsc.parallel_loop(
            0, _NV, unroll=8, carry=(zeros_i, zeros_f))(_compact_body)
    plsc.store_scatter(cx, [pos_l + lane], plsc.bitcast(lo_key, jnp.float32))
    nj = (jnp.max(pos_l) + _L - 1) // _L

    tail_scope = jax.named_scope("ph_tail")
    tail_scope.__enter__()
    low_mask = jnp.int32(0x00FFFFFF)
    prefix = zeros_i
    for step in range(12):
        sh = 22 - 2 * step
        t1 = prefix | (1 << sh)
        t2 = prefix | (2 << sh)
        t3 = prefix | (3 << sh)

        def _cnt_body(j, c, t1=t1, t2=t2, t3=t3):
            c1, c2, c3 = c
            low = _keys(cx[pl.ds(pl.multiple_of(j * _L, _L), _L)]) & low_mask
            c1 = c1 + plsc.all_reduce_population_count(low >= t1)
            c2 = c2 + plsc.all_reduce_population_count(low >= t2)
            c3 = c3 + plsc.all_reduce_population_count(low >= t3)
            return c1, c2, c3

        c1, c2, c3 = lax.fori_loop(
            0, nj, _cnt_body, (zeros_i, zeros_i, zeros_i))
        prefix = jnp.where(c1 >= r0, t1, prefix)
        prefix = jnp.where(c2 >= r0, t2, prefix)
        prefix = jnp.where(c3 >= r0, t3, prefix)

    def _stats_body(j, st):
        c_gt, c_eq, s_gt = st
        xi = cx[pl.ds(pl.multiple_of(j * _L, _L), _L)]
        lowv = _keys(xi) & low_mask
        m_gt = lowv > prefix
        c_gt = c_gt + plsc.all_reduce_population_count(m_gt)
        c_eq = c_eq + plsc.all_reduce_population_count(lowv == prefix)
        s_gt = s_gt + jnp.where(m_gt, xi, 0.0)
        return c_gt, c_eq, s_gt

    c_gt_c, c_eq, s_gt_c = lax.fori_loop(
        0, nj, _stats_body, (zeros_i, zeros_i, zeros_f))
    tail_scope.__exit__(None, None, None)

    count_gt = count_gt_b0 + c_gt_c
    thresh_key = lo_key | prefix
    tbits = jnp.where(thresh_key < 0, thresh_key ^ jnp.int32(0x7FFFFFFF),
                      thresh_key)
    thresh_val = plsc.bitcast(tbits, jnp.float32)

    sum_gt_vec = sum_above + s_gt_c
    sum_gt = jnp.broadcast_to(jnp.sum(sum_gt_vec), (_L,))
    mag = sum_gt + (_splat_i32(_K) - count_gt).astype(jnp.float32) * thresh_val
    denom = mag - mag * jnp.float32(_INV_SQRT2) + jnp.float32(1e-7)
    inv = jnp.float32(1.0) / denom
    inv = jnp.where(jnp.abs(inv) == jnp.inf, 0.0, inv)

    with jax.named_scope("ph_out"):
        @plsc.parallel_loop(0, _NV, unroll=8)
        def _out_body(v):
            xi = xrow[pl.ds(pl.multiple_of(v * _L, _L), _L)]
            orow[pl.ds(pl.multiple_of(v * _L, _L), _L)] = jnp.where(
                xi >= thresh_val, xi * inv, 0.0)

    r_eq = _splat_i32(_K) - count_gt

    @pl.when(jnp.max(c_eq) > jnp.max(r_eq))
    def _tie_fix():
        def _fix_body(v, seen):
            xi = xrow[pl.ds(pl.multiple_of(v * _L, _L), _L)]
            m_eq = _keys(xi) == thresh_key
            mi = jnp.where(m_eq, 1, 0).astype(jnp.int32)
            rank = seen + plsc.cumsum(mi) - mi
            kill = m_eq & (rank >= r_eq)
            ov = orow[pl.ds(pl.multiple_of(v * _L, _L), _L)]
            orow[pl.ds(pl.multiple_of(v * _L, _L), _L)] = jnp.where(
                kill, 0.0, ov)
            return seen + plsc.all_reduce_population_count(m_eq)

        lax.fori_loop(0, _NV, _fix_body, zeros_i)


def _make_kernel():
    mesh = plsc.VectorSubcoreMesh(core_axis_name="c", subcore_axis_name="s")

    @functools.partial(
        pl.kernel,
        out_type=jax.ShapeDtypeStruct((_R, _C), jnp.float32),
        mesh=mesh,
        compiler_params=pltpu.CompilerParams(needs_layout_passes=False),
        scratch_types=[
            pltpu.VMEM((_C,), jnp.float32),
            pltpu.VMEM((_C,), jnp.float32),
            pltpu.VMEM((_C,), jnp.float32),
            pltpu.VMEM((_C,), jnp.float32),
            pltpu.VMEM((16 * 257,), jnp.int32),
            pltpu.VMEM((256,), jnp.int32),
            pltpu.VMEM((_C + _L,), jnp.float32),
            pltpu.SemaphoreType.DMA,
            pltpu.SemaphoreType.DMA,
            pltpu.SemaphoreType.DMA,
            pltpu.SemaphoreType.DMA,
        ],
    )
    def _remaxk(x_hbm, out_hbm, xrow0, xrow1, orow0, orow1, hist, cumh,
                cx, sin0, sin1, sout0, sout1):
        wid = lax.axis_index("s") * _NCORE + lax.axis_index("c")
        row0 = wid * _ROWS_PER_W

        cp0 = pltpu.async_copy(x_hbm.at[row0], xrow0, sin0)
        cp1 = pltpu.async_copy(x_hbm.at[row0 + 1], xrow1, sin1)
        cp0.wait()
        _process_row(xrow0, orow0, hist, cumh, cx)
        w0 = pltpu.async_copy(orow0, out_hbm.at[row0], sout0)
        cp1.wait()
        _process_row(xrow1, orow1, hist, cumh, cx)
        w1 = pltpu.async_copy(orow1, out_hbm.at[row0 + 1], sout1)
        w0.wait()
        w1.wait()

    return _remaxk


_remaxk_kernel = _make_kernel()


@jax.jit
def kernel(x):
    return _remaxk_kernel(x)


# speedup vs baseline: 1.2746x; 1.0008x over previous
import functools

import jax
import jax.numpy as jnp
from jax import lax
from jax.experimental import pallas as pl
from jax.experimental.pallas import tpu as pltpu
from jax.experimental.pallas import tpu_sc as plsc

_R, _C = 64, 8192
_K = 128
_L = 16
_NV = _C // _L
_NCORE, _NSUB = 2, 16
_NW = _NCORE * _NSUB
_ROWS_PER_W = _R // _NW
_CL = 513
_INV_SQRT2 = 0.7071067811865476


def _splat_i32(v):
    return jnp.full((_L,), v, jnp.int32)


def _keys(xi):
    b = plsc.bitcast(xi, jnp.int32)
    return b ^ lax.shift_right_logical(b >> 31, 1)


def _process_row(xrow, orow, hist, cumh, cx):
    lane = lax.iota(jnp.int32, _L)
    lane_base = lane * 257
    ones = jnp.ones((_L,), jnp.int32)
    zeros_i = jnp.zeros((_L,), jnp.int32)
    zeros_f = jnp.zeros((_L,), jnp.float32)

    with jax.named_scope("ph_zero"):
        @plsc.parallel_loop(0, 257, unroll=8)
        def _zero_body(i):
            hist[pl.ds(pl.multiple_of(i * _L, _L), _L)] = zeros_i

    lane_base128 = lane_base + 128
    with jax.named_scope("ph_hist"):
        @plsc.parallel_loop(0, _NV, unroll=8)
        def _hist_body(v):
            xi = xrow[pl.ds(pl.multiple_of(v * _L, _L), _L)]
            plsc.addupdate_scatter(
                hist, [(_keys(xi) >> 24) + lane_base128], ones)

    scan_scope = jax.named_scope("ph_scan")
    scan_scope.__enter__()
    carry = zeros_i
    for c in range(16):
        tot = hist[pl.ds(c * _L, _L)]
        for l in range(1, 16):
            tot = tot + hist[pl.ds(l * 257 + c * _L, _L)]
        pc = plsc.cumsum(tot) + carry
        cumh[pl.ds(c * _L, _L)] = pc
        carry = plsc.load_gather(cumh, [_splat_i32(c * _L + 15)])

    target = _splat_i32(_C - _K)
    b0 = zeros_i
    for c in range(16):
        pc = cumh[pl.ds(c * _L, _L)]
        m = pc <= target
        if c == 15:
            m = m & (lane < 15)
        b0 = b0 + plsc.all_reduce_population_count(m)
    p_b0 = plsc.load_gather(cumh, [b0])
    count_gt_b0 = _splat_i32(_C) - p_b0
    r0 = _splat_i32(_K) - count_gt_b0
    scan_scope.__exit__(None, None, None)

    lo_key = (b0 - 128) << 24
    hi_key = jnp.where(b0 == 255, jnp.int32(0x7F800000),
                       lo_key + jnp.int32(0x01000000))

    def _compact_body(v, st):
        pos, s_above = st
        xi = xrow[pl.ds(pl.multiple_of(v * _L, _L), _L)]
        key = _keys(xi)
        m_gt = key >= hi_key
        s_above = s_above + jnp.where(m_gt, xi, 0.0)
        m_in = (key >= lo_key) & (~m_gt)
        mi = jnp.where(m_in, 1, 0)
        plsc.store_scatter(cx, [pos + plsc.cumsum(mi) - mi], xi, mask=m_in)
        pos = pos + plsc.all_reduce_population_count(m_in)
        return pos, s_above

    with jax.named_scope("ph_compact"):
        pos_l, sum_above = plsc.parallel_loop(
            0, _NV, unroll=8, carry=(zeros_i, zeros_f))(_compact_body)
    plsc.store_scatter(cx, [pos_l + lane], plsc.bitcast(lo_key, jnp.float32))
    nj = (jnp.max(pos_l) + _L - 1) // _L

    tail_scope = jax.named_scope("ph_tail")
    tail_scope.__enter__()
    low_mask = jnp.int32(0x00FFFFFF)
    prefix = zeros_i
    for step in range(12):
        sh = 22 - 2 * step
        t1 = prefix | (1 << sh)
        t2 = prefix | (2 << sh)
        t3 = prefix | (3 << sh)

        def _cnt_body(j, c, t1=t1, t2=t2, t3=t3):
            c1, c2, c3 = c
            low = _keys(cx[pl.ds(pl.multiple_of(j * _L, _L), _L)]) & low_mask
            c1 = c1 + plsc.all_reduce_population_count(low >= t1)
            c2 = c2 + plsc.all_reduce_population_count(low >= t2)
            c3 = c3 + plsc.all_reduce_population_count(low >= t3)
            return c1, c2, c3

        c1, c2, c3 = lax.fori_loop(
            0, nj, _cnt_body, (zeros_i, zeros_i, zeros_i))
        prefix = jnp.where(c1 >= r0, t1, prefix)
        prefix = jnp.where(c2 >= r0, t2, prefix)
        prefix = jnp.where(c3 >= r0, t3, prefix)

    def _stats_body(j, st):
        c_gt, c_eq, s_gt = st
        xi = cx[pl.ds(pl.multiple_of(j * _L, _L), _L)]
        lowv = _keys(xi) & low_mask
        m_gt = lowv > prefix
        c_gt = c_gt + plsc.all_reduce_population_count(m_gt)
        c_eq = c_eq + plsc.all_reduce_population_count(lowv == prefix)
        s_gt = s_gt + jnp.where(m_gt, xi, 0.0)
        return c_gt, c_eq, s_gt

    c_gt_c, c_eq, s_gt_c = lax.fori_loop(
        0, nj, _stats_body, (zeros_i, zeros_i, zeros_f))
    tail_scope.__exit__(None, None, None)

    count_gt = count_gt_b0 + c_gt_c
    thresh_key = lo_key | prefix
    tbits = jnp.where(thresh_key < 0, thresh_key ^ jnp.int32(0x7FFFFFFF),
                      thresh_key)
    thresh_val = plsc.bitcast(tbits, jnp.float32)

    sum_gt_vec = sum_above + s_gt_c
    sum_gt = jnp.broadcast_to(jnp.sum(sum_gt_vec), (_L,))
    mag = sum_gt + (_splat_i32(_K) - count_gt).astype(jnp.float32) * thresh_val
    denom = mag - mag * jnp.float32(_INV_SQRT2) + jnp.float32(1e-7)
    inv = jnp.float32(1.0) / denom
    inv = jnp.where(jnp.abs(inv) == jnp.inf, 0.0, inv)

    with jax.named_scope("ph_out"):
        @plsc.parallel_loop(0, _NV, unroll=8)
        def _out_body(v):
            xi = xrow[pl.ds(pl.multiple_of(v * _L, _L), _L)]
            orow[pl.ds(pl.multiple_of(v * _L, _L), _L)] = jnp.where(
                xi >= thresh_val, xi * inv, 0.0)

    r_eq = _splat_i32(_K) - count_gt

    @pl.when(jnp.max(c_eq) > jnp.max(r_eq))
    def _tie_fix():
        def _fix_body(v, seen):
            xi = xrow[pl.ds(pl.multiple_of(v * _L, _L), _L)]
            m_eq = _keys(xi) == thresh_key
            mi = jnp.where(m_eq, 1, 0).astype(jnp.int32)
            rank = seen + plsc.cumsum(mi) - mi
            kill = m_eq & (rank >= r_eq)
            ov = orow[pl.ds(pl.multiple_of(v * _L, _L), _L)]
            orow[pl.ds(pl.multiple_of(v * _L, _L), _L)] = jnp.where(
                kill, 0.0, ov)
            return seen + plsc.all_reduce_population_count(m_eq)

        lax.fori_loop(0, _NV, _fix_body, zeros_i)


def _make_kernel():
    mesh = plsc.VectorSubcoreMesh(core_axis_name="c", subcore_axis_name="s")

    @functools.partial(
        pl.kernel,
        out_type=jax.ShapeDtypeStruct((_R, _C), jnp.float32),
        mesh=mesh,
        compiler_params=pltpu.CompilerParams(
            needs_layout_passes=False,
            disable_bounds_checks=True,
            disable_semaphore_checks=True,
        ),
        scratch_types=[
            pltpu.VMEM((_C,), jnp.float32),
            pltpu.VMEM((_C,), jnp.float32),
            pltpu.VMEM((_C,), jnp.float32),
            pltpu.VMEM((_C,), jnp.float32),
            pltpu.VMEM((16 * 257,), jnp.int32),
            pltpu.VMEM((256,), jnp.int32),
            pltpu.VMEM((_C + _L,), jnp.float32),
            pltpu.SemaphoreType.DMA,
            pltpu.SemaphoreType.DMA,
            pltpu.SemaphoreType.DMA,
            pltpu.SemaphoreType.DMA,
        ],
    )
    def _remaxk(x_hbm, out_hbm, xrow0, xrow1, orow0, orow1, hist, cumh,
                cx, sin0, sin1, sout0, sout1):
        wid = lax.axis_index("s") * _NCORE + lax.axis_index("c")
        row0 = wid * _ROWS_PER_W

        cp0 = pltpu.async_copy(x_hbm.at[row0], xrow0, sin0)
        cp1 = pltpu.async_copy(x_hbm.at[row0 + 1], xrow1, sin1)
        cp0.wait()
        _process_row(xrow0, orow0, hist, cumh, cx)
        w0 = pltpu.async_copy(orow0, out_hbm.at[row0], sout0)
        cp1.wait()
        _process_row(xrow1, orow1, hist, cumh, cx)
        w1 = pltpu.async_copy(orow1, out_hbm.at[row0 + 1], sout1)
        w0.wait()
        w1.wait()

    return _remaxk


_remaxk_kernel = _make_kernel()


@jax.jit
def kernel(x):
    return _remaxk_kernel(x)


# speedup vs baseline: 1.2866x; 1.0094x over previous
import functools

import jax
import jax.numpy as jnp
from jax import lax
from jax.experimental import pallas as pl
from jax.experimental.pallas import tpu as pltpu
from jax.experimental.pallas import tpu_sc as plsc

_R, _C = 64, 8192
_K = 128
_L = 16
_NV = _C // _L
_NCORE, _NSUB = 2, 16
_NW = _NCORE * _NSUB
_ROWS_PER_W = _R // _NW
_CL = 513
_INV_SQRT2 = 0.7071067811865476


def _splat_i32(v):
    return jnp.full((_L,), v, jnp.int32)


def _keys(xi):
    b = plsc.bitcast(xi, jnp.int32)
    return b ^ lax.shift_right_logical(b >> 31, 1)


def _zero_hist(hist):
    zeros_i = jnp.zeros((_L,), jnp.int32)

    @plsc.parallel_loop(0, 257, unroll=8)
    def _zero_body(i):
        hist[pl.ds(pl.multiple_of(i * _L, _L), _L)] = zeros_i


def _process_row(xrow, orow, hist, cumh, cx):
    lane = lax.iota(jnp.int32, _L)
    lane_base = lane * 257
    ones = jnp.ones((_L,), jnp.int32)
    zeros_i = jnp.zeros((_L,), jnp.int32)
    zeros_f = jnp.zeros((_L,), jnp.float32)

    lane_base128 = lane_base + 128

    @plsc.parallel_loop(0, _NV, unroll=8)
    def _hist_body(v):
        xi = xrow[pl.ds(pl.multiple_of(v * _L, _L), _L)]
        plsc.addupdate_scatter(
            hist, [(_keys(xi) >> 24) + lane_base128], ones)

    carry = zeros_i
    for c in range(16):
        tot = hist[pl.ds(c * _L, _L)]
        for l in range(1, 16):
            tot = tot + hist[pl.ds(l * 257 + c * _L, _L)]
        pc = plsc.cumsum(tot) + carry
        cumh[pl.ds(c * _L, _L)] = pc
        carry = plsc.load_gather(cumh, [_splat_i32(c * _L + 15)])

    target = _splat_i32(_C - _K)
    b0 = zeros_i
    for c in range(16):
        pc = cumh[pl.ds(c * _L, _L)]
        m = pc <= target
        if c == 15:
            m = m & (lane < 15)
        b0 = b0 + plsc.all_reduce_population_count(m)
    p_b0 = plsc.load_gather(cumh, [b0])
    count_gt_b0 = _splat_i32(_C) - p_b0
    r0 = _splat_i32(_K) - count_gt_b0

    lo_key = (b0 - 128) << 24
    hi_key = jnp.where(b0 == 255, jnp.int32(0x7F800000),
                       lo_key + jnp.int32(0x01000000))

    def _compact_body(v, st):
        pos, s_above = st
        xi = xrow[pl.ds(pl.multiple_of(v * _L, _L), _L)]
        key = _keys(xi)
        m_gt = key >= hi_key
        s_above = s_above + jnp.where(m_gt, xi, 0.0)
        m_in = (key >= lo_key) & (~m_gt)
        mi = jnp.where(m_in, 1, 0)
        plsc.store_scatter(cx, [pos + plsc.cumsum(mi) - mi], xi, mask=m_in)
        pos = pos + plsc.all_reduce_population_count(m_in)
        return pos, s_above

    pos_l, sum_above = plsc.parallel_loop(
        0, _NV, unroll=8, carry=(zeros_i, zeros_f))(_compact_body)
    plsc.store_scatter(cx, [pos_l + lane], plsc.bitcast(lo_key, jnp.float32))
    nj = (jnp.max(pos_l) + _L - 1) // _L

    low_mask = jnp.int32(0x00FFFFFF)
    prefix = zeros_i
    for step in range(12):
        sh = 22 - 2 * step
        t1 = prefix | (1 << sh)
        t2 = prefix | (2 << sh)
        t3 = prefix | (3 << sh)

        def _cnt_body(j, c, t1=t1, t2=t2, t3=t3):
            c1, c2, c3 = c
            low = _keys(cx[pl.ds(pl.multiple_of(j * _L, _L), _L)]) & low_mask
            c1 = c1 + plsc.all_reduce_population_count(low >= t1)
            c2 = c2 + plsc.all_reduce_population_count(low >= t2)
            c3 = c3 + plsc.all_reduce_population_count(low >= t3)
            return c1, c2, c3

        c1, c2, c3 = lax.fori_loop(
            0, nj, _cnt_body, (zeros_i, zeros_i, zeros_i))
        prefix = jnp.where(c1 >= r0, t1, prefix)
        prefix = jnp.where(c2 >= r0, t2, prefix)
        prefix = jnp.where(c3 >= r0, t3, prefix)

    def _stats_body(j, st):
        c_gt, c_eq, s_gt = st
        xi = cx[pl.ds(pl.multiple_of(j * _L, _L), _L)]
        lowv = _keys(xi) & low_mask
        m_gt = lowv > prefix
        c_gt = c_gt + plsc.all_reduce_population_count(m_gt)
        c_eq = c_eq + plsc.all_reduce_population_count(lowv == prefix)
        s_gt = s_gt + jnp.where(m_gt, xi, 0.0)
        return c_gt, c_eq, s_gt

    c_gt_c, c_eq, s_gt_c = lax.fori_loop(
        0, nj, _stats_body, (zeros_i, zeros_i, zeros_f))

    count_gt = count_gt_b0 + c_gt_c
    thresh_key = lo_key | prefix
    tbits = jnp.where(thresh_key < 0, thresh_key ^ jnp.int32(0x7FFFFFFF),
                      thresh_key)
    thresh_val = plsc.bitcast(tbits, jnp.float32)

    sum_gt_vec = sum_above + s_gt_c
    sum_gt = jnp.broadcast_to(jnp.sum(sum_gt_vec), (_L,))
    mag = sum_gt + (_splat_i32(_K) - count_gt).astype(jnp.float32) * thresh_val
    denom = mag - mag * jnp.float32(_INV_SQRT2) + jnp.float32(1e-7)
    inv = jnp.float32(1.0) / denom
    inv = jnp.where(jnp.abs(inv) == jnp.inf, 0.0, inv)

    @plsc.parallel_loop(0, _NV, unroll=8)
    def _out_body(v):
        xi = xrow[pl.ds(pl.multiple_of(v * _L, _L), _L)]
        orow[pl.ds(pl.multiple_of(v * _L, _L), _L)] = jnp.where(
            xi >= thresh_val, xi * inv, 0.0)

    r_eq = _splat_i32(_K) - count_gt

    @pl.when(jnp.max(c_eq) > jnp.max(r_eq))
    def _tie_fix():
        def _fix_body(v, seen):
            xi = xrow[pl.ds(pl.multiple_of(v * _L, _L), _L)]
            m_eq = _keys(xi) == thresh_key
            mi = jnp.where(m_eq, 1, 0).astype(jnp.int32)
            rank = seen + plsc.cumsum(mi) - mi
            kill = m_eq & (rank >= r_eq)
            ov = orow[pl.ds(pl.multiple_of(v * _L, _L), _L)]
            orow[pl.ds(pl.multiple_of(v * _L, _L), _L)] = jnp.where(
                kill, 0.0, ov)
            return seen + plsc.all_reduce_population_count(m_eq)

        lax.fori_loop(0, _NV, _fix_body, zeros_i)


def _make_kernel():
    mesh = plsc.VectorSubcoreMesh(core_axis_name="c", subcore_axis_name="s")

    @functools.partial(
        pl.kernel,
        out_type=jax.ShapeDtypeStruct((_R, _C), jnp.float32),
        mesh=mesh,
        compiler_params=pltpu.CompilerParams(needs_layout_passes=False),
        scratch_types=[
            pltpu.VMEM((_C,), jnp.float32),
            pltpu.VMEM((_C,), jnp.float32),
            pltpu.VMEM((_C,), jnp.float32),
            pltpu.VMEM((_C,), jnp.float32),
            pltpu.VMEM((16 * 257,), jnp.int32),
            pltpu.VMEM((256,), jnp.int32),
            pltpu.VMEM((_C + _L,), jnp.float32),
            pltpu.SemaphoreType.DMA,
            pltpu.SemaphoreType.DMA,
            pltpu.SemaphoreType.DMA,
            pltpu.SemaphoreType.DMA,
        ],
    )
    def _remaxk(x_hbm, out_hbm, xrow0, xrow1, orow0, orow1, hist, cumh,
                cx, sin0, sin1, sout0, sout1):
        wid = lax.axis_index("s") * _NCORE + lax.axis_index("c")
        row0 = wid * _ROWS_PER_W

        cp0 = pltpu.async_copy(x_hbm.at[row0], xrow0, sin0)
        cp1 = pltpu.async_copy(x_hbm.at[row0 + 1], xrow1, sin1)
        _zero_hist(hist)
        cp0.wait()
        _process_row(xrow0, orow0, hist, cumh, cx)
        w0 = pltpu.async_copy(orow0, out_hbm.at[row0], sout0)
        _zero_hist(hist)
        cp1.wait()
        _process_row(xrow1, orow1, hist, cumh, cx)
        w1 = pltpu.async_copy(orow1, out_hbm.at[row0 + 1], sout1)
        w0.wait()
        w1.wait()

    return _remaxk


_remaxk_kernel = _make_kernel()


@jax.jit
def kernel(x):
    return _remaxk_kernel(x)
